# baseline (speedup 1.0000x reference)
import functools

import numpy as np
import jax
import jax.numpy as jnp
from jax import lax
from jax.experimental import pallas as pl
from jax.experimental.pallas import tpu as pltpu
from jax.experimental.pallas import tpu_sc as plsc

MAX_SEQ = 4096
D_MODEL = 1024
BATCH = 4
SEQ = 4096

_INFO = plsc.get_sparse_core_info()
NC, NS, L = _INFO.num_cores, _INFO.num_subcores, _INFO.num_lanes
NW = NC * NS
POS_PER_W = SEQ // NW
CHUNK = 32
NCH = POS_PER_W // CHUNK
NUNIT = NCH * BATCH
NSLOT = 2
PCH = 16
NPQ = POS_PER_W // PCH
NPBUF = 3


def _make_pe_np() -> np.ndarray:
    pos = np.arange(MAX_SEQ, dtype=np.float32)[:, None]
    i = np.arange(D_MODEL, dtype=np.float32)[None, :]
    angles = pos / np.power(10000.0, 2.0 * np.floor(i / 2.0) / D_MODEL)
    even = (np.arange(D_MODEL) % 2 == 0)[None, :]
    pe = np.where(even, np.sin(angles), np.cos(angles))
    return pe.astype(np.float32)


_PE_NP = _make_pe_np()

_mesh = plsc.VectorSubcoreMesh(core_axis_name="c", subcore_axis_name="s")


@functools.partial(
    pl.kernel,
    out_type=jax.ShapeDtypeStruct((BATCH * SEQ, D_MODEL), jnp.float32),
    mesh=_mesh,
    scratch_types=[
        pltpu.VMEM((BATCH, POS_PER_W), jnp.int32),
        pltpu.VMEM((NPBUF, PCH, D_MODEL), jnp.float32),
        pltpu.VMEM((NSLOT, CHUNK, D_MODEL), jnp.float32),
        pltpu.SemaphoreType.DMA((NSLOT,)),
        pltpu.SemaphoreType.DMA((NPBUF,)),
        pltpu.SemaphoreType.DMA,
    ],
)
def _embed_sc(table_hbm, idx_hbm, pe_hbm, out_hbm, idx_v, pe_v, rows_v,
              sems, psems, isem):
    wid = lax.axis_index("s") * NC + lax.axis_index("c")
    pos0 = wid * POS_PER_W

    def _mod(v, m):
        return v % m if isinstance(v, int) else lax.rem(v, m)

    def issue_pe(q):
        pltpu.async_copy(
            pe_hbm.at[pl.ds(pos0 + q * PCH, PCH)], pe_v.at[_mod(q, NPBUF)],
            psems.at[_mod(q, NPBUF)])

    def wait_pe(q):
        pltpu.make_async_copy(
            pe_hbm.at[pl.ds(0, PCH)], pe_v.at[0],
            psems.at[_mod(q, NPBUF)]).wait()

    def issue_gather(u):
        b, slot = _mod(u, BATCH), _mod(u, NSLOT)
        c = u >> 2 if isinstance(u, int) else lax.shift_right_logical(u, 2)
        pltpu.async_copy(
            table_hbm.at[idx_v.at[b, pl.ds(c * CHUNK, CHUNK)]],
            rows_v.at[slot], sems.at[slot])

    def wait_gather(slot):
        pltpu.make_async_copy(
            table_hbm.at[idx_v.at[0, pl.ds(0, CHUNK)]], rows_v.at[0],
            sems.at[slot]).wait()

    def wait_out(slot):
        pltpu.make_async_copy(
            rows_v.at[0], out_hbm.at[pl.ds(0, CHUNK)], sems.at[slot]).wait()

    for q in range(NPBUF):
        issue_pe(q)
    for b in range(BATCH):
        pltpu.async_copy(
            idx_hbm.at[b, pl.ds(pos0, POS_PER_W)], idx_v.at[b], isem)
    for b in range(BATCH):
        pltpu.make_async_copy(
            idx_hbm.at[0, pl.ds(0, POS_PER_W)], idx_v.at[0], isem).wait()
    issue_gather(0)

    def unit_body(u, _):
        b = u & 3
        slot = u & 1
        c = lax.shift_right_logical(u, 2)
        q0 = 2 * c
        q0b = lax.rem(q0, NPBUF)
        q1b = lax.rem(q0 + 1, NPBUF)

        @pl.when(u + 1 < NUNIT)
        def _():
            @pl.when(u >= 1)
            def _():
                wait_out(slot ^ 1)

            issue_gather(u + 1)

        @pl.when(b == 0)
        def _():
            wait_pe(q0)
            wait_pe(q0 + 1)

        wait_gather(slot)

        @plsc.parallel_loop(0, PCH)
        def add_lo(i):
            for j in range(D_MODEL // L):
                sl = pl.ds(j * L, L)
                plsc.addupdate(rows_v.at[slot, i, sl], pe_v[q0b, i, sl])

        @plsc.parallel_loop(0, PCH)
        def add_hi(i):
            for j in range(D_MODEL // L):
                sl = pl.ds(j * L, L)
                plsc.addupdate(rows_v.at[slot, PCH + i, sl], pe_v[q1b, i, sl])

        @pl.when(b == 3)
        def _():
            @pl.when(q0 + 3 < NPQ)
            def _():
                issue_pe(q0 + 3)

            @pl.when(q0 + 4 < NPQ)
            def _():
                issue_pe(q0 + 4)

        orow = b * SEQ + pos0 + c * CHUNK
        pltpu.async_copy(
            rows_v.at[slot], out_hbm.at[pl.ds(orow, CHUNK)], sems.at[slot])
        return 0

    lax.fori_loop(0, NUNIT, unit_body, 0)
    wait_out((NUNIT - 2) % NSLOT)
    wait_out((NUNIT - 1) % NSLOT)


def kernel(x, token_table):
    pe = jnp.asarray(_PE_NP)
    out = _embed_sc(token_table, x.astype(jnp.int32), pe)
    return out.reshape(BATCH, SEQ, D_MODEL)


# speedup vs baseline: 1.2128x; 1.2128x over previous
import functools

import numpy as np
import jax
import jax.numpy as jnp
from jax import lax
from jax.experimental import pallas as pl
from jax.experimental.pallas import tpu as pltpu
from jax.experimental.pallas import tpu_sc as plsc

MAX_SEQ = 4096
D_MODEL = 1024
BATCH = 4
SEQ = 4096

_INFO = plsc.get_sparse_core_info()
NC, NS, L = _INFO.num_cores, _INFO.num_subcores, _INFO.num_lanes
NW = NC * NS
POS_PER_W = SEQ // NW
CHUNK = 16
NCH = POS_PER_W // CHUNK
NUNIT = NCH * BATCH
NSLOT = 4


def _make_pe_np() -> np.ndarray:
    pos = np.arange(MAX_SEQ, dtype=np.float32)[:, None]
    i = np.arange(D_MODEL, dtype=np.float32)[None, :]
    angles = pos / np.power(10000.0, 2.0 * np.floor(i / 2.0) / D_MODEL)
    even = (np.arange(D_MODEL) % 2 == 0)[None, :]
    pe = np.where(even, np.sin(angles), np.cos(angles))
    return pe.astype(np.float32)


_PE_NP = _make_pe_np()

_mesh = plsc.VectorSubcoreMesh(core_axis_name="c", subcore_axis_name="s")


@functools.partial(
    pl.kernel,
    out_type=jax.ShapeDtypeStruct((BATCH * SEQ, D_MODEL), jnp.float32),
    mesh=_mesh,
    scratch_types=[
        pltpu.VMEM((BATCH, POS_PER_W), jnp.int32),
        pltpu.VMEM((2, CHUNK, D_MODEL), jnp.float32),
        pltpu.VMEM((NSLOT, CHUNK, D_MODEL), jnp.float32),
        pltpu.SemaphoreType.DMA((NSLOT,)),
        pltpu.SemaphoreType.DMA,
    ],
)
def _embed_sc(table_hbm, idx_hbm, pe_hbm, out_hbm, idx_v, pe_v, rows_v,
              sems, psem):
    wid = lax.axis_index("s") * NC + lax.axis_index("c")
    pos0 = wid * POS_PER_W

    for b in range(BATCH):
        pltpu.async_copy(
            idx_hbm.at[b, pl.ds(pos0, POS_PER_W)], idx_v.at[b], psem)

    def issue_gather(u):
        b = u & 3
        c = lax.shift_right_logical(u, 2)
        pltpu.async_copy(
            table_hbm.at[idx_v.at[b, pl.ds(c * CHUNK, CHUNK)]],
            rows_v.at[u & 3], sems.at[u & 3])

    def issue_pe(c):
        pltpu.async_copy(
            pe_hbm.at[pl.ds(pos0 + c * CHUNK, CHUNK)], pe_v.at[c & 1], psem)

    def wait_out(slot):
        pltpu.make_async_copy(
            rows_v.at[0], out_hbm.at[pl.ds(0, CHUNK)], sems.at[slot]).wait()

    def wait_gather(slot):
        pltpu.make_async_copy(
            table_hbm.at[idx_v.at[0, pl.ds(0, CHUNK)]], rows_v.at[0],
            sems.at[slot]).wait()

    def wait_pe():
        pltpu.make_async_copy(
            pe_hbm.at[pl.ds(0, CHUNK)], pe_v.at[0], psem).wait()

    for _ in range(BATCH):
        pltpu.make_async_copy(
            idx_hbm.at[0, pl.ds(0, POS_PER_W)], idx_v.at[0], psem).wait()
    issue_pe(0)
    issue_gather(0)
    issue_gather(1)

    def unit_body(u, _):
        slot = u & 3
        c = lax.shift_right_logical(u, 2)
        s2 = (u + 2) & 3

        @pl.when(u >= 2)
        def _():
            wait_out(s2)

        @pl.when(u + 2 < NUNIT)
        def _():
            issue_gather(u + 2)

        @pl.when(slot == 0)
        def _():
            wait_pe()

            @pl.when(c + 1 < NCH)
            def _():
                issue_pe(c + 1)

        wait_gather(slot)

        pec = c & 1

        @plsc.parallel_loop(0, CHUNK)
        def add_body(i):
            for j in range(D_MODEL // L):
                sl = pl.ds(j * L, L)
                plsc.addupdate(rows_v.at[slot, i, sl], pe_v[pec, i, sl])

        orow = slot * SEQ + pos0 + c * CHUNK
        pltpu.async_copy(
            rows_v.at[slot], out_hbm.at[pl.ds(orow, CHUNK)], sems.at[slot])
        return 0

    lax.fori_loop(0, NUNIT, unit_body, 0)
    wait_out(2)
    wait_out(3)


def kernel(x, token_table):
    pe = jnp.asarray(_PE_NP)
    out = _embed_sc(token_table, x.astype(jnp.int32), pe)
    return out.reshape(BATCH, SEQ, D_MODEL)
